# baseline (speedup 1.0000x reference)
import functools

import jax
import jax.numpy as jnp
from jax import lax
from jax.experimental import pallas as pl
from jax.experimental.pallas import tpu as pltpu
from jax.experimental.pallas import tpu_sc as plsc

_B, _S, _D = 4, 8192, 1024
_S_BLK = 2048
_TC_B = 3
_NC, _NS = 2, 16
_NW = _NC * _NS
_ROWS_W = _S // _NW
_R = 16
_NCH = _ROWS_W // _R
_LANES = 16
_VPR = _D // _LANES

_mesh = plsc.VectorSubcoreMesh(core_axis_name="c", subcore_axis_name="s")


def _add_body(x_ref, pos_ref, out_ref):
    out_ref[...] = x_ref[...] + pos_ref[...][None, :, :]


def _tc_part(x, pos_embedding):
    return pl.pallas_call(
        _add_body,
        grid=(_S // _S_BLK, _TC_B),
        in_specs=[
            pl.BlockSpec((1, _S_BLK, _D), lambda i, b: (b, i, 0)),
            pl.BlockSpec((_S_BLK, _D), lambda i, b: (i, 0)),
        ],
        out_specs=pl.BlockSpec((1, _S_BLK, _D), lambda i, b: (b, i, 0)),
        out_shape=jax.ShapeDtypeStruct((_TC_B, _S, _D), x.dtype),
    )(x, pos_embedding)


@functools.partial(
    pl.kernel,
    mesh=_mesh,
    out_type=jax.ShapeDtypeStruct((_S, _D), jnp.float32),
    scratch_types=[
        pltpu.VMEM((_R, _D), jnp.float32),
        pltpu.VMEM((_R, _D), jnp.float32),
    ],
)
def _sc_part(x_hbm, pos_hbm, out_hbm, pos_v, x_v):
    wid = lax.axis_index("s") * _NC + lax.axis_index("c")
    base = wid * _ROWS_W

    def chunk_body(c, carry):
        row0 = base + c * _R
        pltpu.sync_copy(pos_hbm.at[pl.ds(row0, _R)], pos_v)
        pltpu.sync_copy(x_hbm.at[_B - 1, pl.ds(row0, _R)], x_v)

        def row_body(r, carry2):
            for j in range(_VPR):
                sl = pl.ds(j * _LANES, _LANES)
                x_v[r, sl] = x_v[r, sl] + pos_v[r, sl]
            return carry2

        lax.fori_loop(0, _R, row_body, 0)
        pltpu.sync_copy(x_v, out_hbm.at[pl.ds(row0, _R)])
        return carry

    lax.fori_loop(0, _NCH, chunk_body, 0)


def kernel(x, pos_embedding):
    tc_out = _tc_part(x, pos_embedding)
    sc_out = _sc_part(x, pos_embedding)
    return jnp.concatenate([tc_out, sc_out[None]], axis=0)


# speedup vs baseline: 2.3814x; 2.3814x over previous
import jax
import jax.numpy as jnp
from jax.experimental import pallas as pl
from jax.experimental.pallas import tpu as pltpu

_S_BLK = 2048


def _add_body(x_ref, pos_ref, out_ref):
    out_ref[...] = x_ref[...] + pos_ref[...][None, :, :]


def kernel(x, pos_embedding):
    batch, seq_len, d_model = x.shape
    grid = (seq_len // _S_BLK, batch)
    return pl.pallas_call(
        _add_body,
        grid=grid,
        in_specs=[
            pl.BlockSpec((1, _S_BLK, d_model), lambda i, b: (b, i, 0)),
            pl.BlockSpec((_S_BLK, d_model), lambda i, b: (i, 0)),
        ],
        out_specs=pl.BlockSpec((1, _S_BLK, d_model), lambda i, b: (b, i, 0)),
        out_shape=jax.ShapeDtypeStruct(x.shape, x.dtype),
    )(x, pos_embedding)
